# baseline (speedup 1.0000x reference)
import functools

import jax
import jax.numpy as jnp
from jax import lax
from jax.experimental import pallas as pl
from jax.experimental.pallas import tpu as pltpu

T = 2048
D = 1024
E = 8
K = 2
F = 1536
FS = 4096

TT = 256
TF = 512
TFS = 512


def _routed_body(x_ref, router_ref, w1_ref, w3_ref, w2_ref, out_ref, comb_ref):
    e = pl.program_id(1)
    f = pl.program_id(2)
    first = (e == 0) & (f == 0)

    @pl.when(first)
    def _():
        scores = jnp.dot(x_ref[...], router_ref[...],
                         preferred_element_type=jnp.float32)
        lane = lax.broadcasted_iota(jnp.int32, scores.shape, 1)
        m1 = jnp.max(scores, axis=1, keepdims=True)
        i1 = jnp.min(jnp.where(scores == m1, lane, E), axis=1, keepdims=True)
        mask1 = lane == i1
        scores2 = jnp.where(mask1, -jnp.inf, scores)
        m2 = jnp.max(scores2, axis=1, keepdims=True)
        i2 = jnp.min(jnp.where(scores2 == m2, lane, E), axis=1, keepdims=True)
        mask2 = lane == i2
        comb = (jax.nn.sigmoid(m1) * mask1.astype(jnp.float32)
                + jax.nn.sigmoid(m2) * mask2.astype(jnp.float32))
        comb_ref[...] = comb
        out_ref[...] = jnp.zeros_like(out_ref)

    lane = lax.broadcasted_iota(jnp.int32, (TT, E), 1)
    w_col = jnp.sum(comb_ref[...] * (lane == e).astype(jnp.float32),
                    axis=1, keepdims=True)

    x = x_ref[...]
    h1 = lax.dot_general(x, w1_ref[0], (((1,), (1,)), ((), ())),
                         preferred_element_type=jnp.float32)
    h3 = lax.dot_general(x, w3_ref[0], (((1,), (1,)), ((), ())),
                         preferred_element_type=jnp.float32)
    act = (h1 * jax.nn.sigmoid(h1)) * h3 * w_col
    out_ref[...] += lax.dot_general(act, w2_ref[0], (((1,), (1,)), ((), ())),
                                    preferred_element_type=jnp.float32)


def _shared_body(x_ref, routed_ref, w1s_ref, w3s_ref, w2s_ref, out_ref):
    fs = pl.program_id(1)

    @pl.when(fs == 0)
    def _():
        out_ref[...] = routed_ref[...]

    x = x_ref[...]
    h1 = lax.dot_general(x, w1s_ref[...], (((1,), (1,)), ((), ())),
                         preferred_element_type=jnp.float32)
    h3 = lax.dot_general(x, w3s_ref[...], (((1,), (1,)), ((), ())),
                         preferred_element_type=jnp.float32)
    act = (h1 * jax.nn.sigmoid(h1)) * h3
    out_ref[...] += lax.dot_general(act, w2s_ref[...], (((1,), (1,)), ((), ())),
                                    preferred_element_type=jnp.float32)


@jax.jit
def kernel(x, router_DE, w13, w2, w13_shared, w2_shared):
    nt = T // TT
    nf = F // TF
    nfs = FS // TFS

    routed = pl.pallas_call(
        _routed_body,
        grid=(nt, E, nf),
        in_specs=[
            pl.BlockSpec((TT, D), lambda t, e, f: (t, 0)),
            pl.BlockSpec((D, E), lambda t, e, f: (0, 0)),
            pl.BlockSpec((1, TF, D), lambda t, e, f: (e, f, 0)),
            pl.BlockSpec((1, TF, D), lambda t, e, f: (e, nf + f, 0)),
            pl.BlockSpec((1, D, TF), lambda t, e, f: (e, 0, f)),
        ],
        out_specs=pl.BlockSpec((TT, D), lambda t, e, f: (t, 0)),
        out_shape=jax.ShapeDtypeStruct((T, D), jnp.float32),
        scratch_shapes=[pltpu.VMEM((TT, E), jnp.float32)],
        compiler_params=pltpu.CompilerParams(
            dimension_semantics=("parallel", "arbitrary", "arbitrary")),
    )(x, router_DE, w13, w13, w2)

    out = pl.pallas_call(
        _shared_body,
        grid=(nt, nfs),
        in_specs=[
            pl.BlockSpec((TT, D), lambda t, f: (t, 0)),
            pl.BlockSpec((TT, D), lambda t, f: (t, 0)),
            pl.BlockSpec((TFS, D), lambda t, f: (f, 0)),
            pl.BlockSpec((TFS, D), lambda t, f: (nfs + f, 0)),
            pl.BlockSpec((D, TFS), lambda t, f: (0, f)),
        ],
        out_specs=pl.BlockSpec((TT, D), lambda t, f: (t, 0)),
        out_shape=jax.ShapeDtypeStruct((T, D), jnp.float32),
        compiler_params=pltpu.CompilerParams(
            dimension_semantics=("parallel", "arbitrary")),
    )(x, routed, w13_shared, w13_shared, w2_shared)

    return out


# speedup vs baseline: 2.1451x; 2.1451x over previous
import functools

import jax
import jax.numpy as jnp
from jax import lax
from jax.experimental import pallas as pl
from jax.experimental.pallas import tpu as pltpu

T = 2048
D = 1024
E = 8
K = 2
F = 1536
FS = 4096

TT = 2048
TF = 512
TFS = 512


def _routed_body(x_ref, router_ref, w1_ref, w3_ref, w2_ref, out_ref, comb_ref):
    e = pl.program_id(1)
    f = pl.program_id(2)
    first = (e == 0) & (f == 0)

    @pl.when(first)
    def _():
        scores = jnp.dot(x_ref[...], router_ref[...],
                         preferred_element_type=jnp.float32)
        lane = lax.broadcasted_iota(jnp.int32, scores.shape, 1)
        m1 = jnp.max(scores, axis=1, keepdims=True)
        i1 = jnp.min(jnp.where(scores == m1, lane, E), axis=1, keepdims=True)
        mask1 = lane == i1
        scores2 = jnp.where(mask1, -jnp.inf, scores)
        m2 = jnp.max(scores2, axis=1, keepdims=True)
        i2 = jnp.min(jnp.where(scores2 == m2, lane, E), axis=1, keepdims=True)
        mask2 = lane == i2
        comb = (jax.nn.sigmoid(m1) * mask1.astype(jnp.float32)
                + jax.nn.sigmoid(m2) * mask2.astype(jnp.float32))
        comb_ref[...] = comb
        out_ref[...] = jnp.zeros_like(out_ref)

    lane = lax.broadcasted_iota(jnp.int32, (TT, E), 1)
    w_col = jnp.sum(comb_ref[...] * (lane == e).astype(jnp.float32),
                    axis=1, keepdims=True)

    x = x_ref[...]
    h1 = lax.dot_general(x, w1_ref[0], (((1,), (1,)), ((), ())),
                         preferred_element_type=jnp.float32)
    h3 = lax.dot_general(x, w3_ref[0], (((1,), (1,)), ((), ())),
                         preferred_element_type=jnp.float32)
    act = (h1 * jax.nn.sigmoid(h1)) * h3 * w_col
    out_ref[...] += lax.dot_general(act, w2_ref[0], (((1,), (1,)), ((), ())),
                                    preferred_element_type=jnp.float32)


def _shared_body(x_ref, routed_ref, w1s_ref, w3s_ref, w2s_ref, out_ref):
    fs = pl.program_id(1)

    @pl.when(fs == 0)
    def _():
        out_ref[...] = routed_ref[...]

    x = x_ref[...]
    h1 = lax.dot_general(x, w1s_ref[...], (((1,), (1,)), ((), ())),
                         preferred_element_type=jnp.float32)
    h3 = lax.dot_general(x, w3s_ref[...], (((1,), (1,)), ((), ())),
                         preferred_element_type=jnp.float32)
    act = (h1 * jax.nn.sigmoid(h1)) * h3
    out_ref[...] += lax.dot_general(act, w2s_ref[...], (((1,), (1,)), ((), ())),
                                    preferred_element_type=jnp.float32)


@jax.jit
def kernel(x, router_DE, w13, w2, w13_shared, w2_shared):
    nt = T // TT
    nf = F // TF
    nfs = FS // TFS

    routed = pl.pallas_call(
        _routed_body,
        grid=(nt, E, nf),
        in_specs=[
            pl.BlockSpec((TT, D), lambda t, e, f: (t, 0)),
            pl.BlockSpec((D, E), lambda t, e, f: (0, 0)),
            pl.BlockSpec((1, TF, D), lambda t, e, f: (e, f, 0)),
            pl.BlockSpec((1, TF, D), lambda t, e, f: (e, nf + f, 0)),
            pl.BlockSpec((1, D, TF), lambda t, e, f: (e, 0, f)),
        ],
        out_specs=pl.BlockSpec((TT, D), lambda t, e, f: (t, 0)),
        out_shape=jax.ShapeDtypeStruct((T, D), jnp.float32),
        scratch_shapes=[pltpu.VMEM((TT, E), jnp.float32)],
        compiler_params=pltpu.CompilerParams(
            dimension_semantics=("parallel", "arbitrary", "arbitrary")),
    )(x, router_DE, w13, w13, w2)

    out = pl.pallas_call(
        _shared_body,
        grid=(nt, nfs),
        in_specs=[
            pl.BlockSpec((TT, D), lambda t, f: (t, 0)),
            pl.BlockSpec((TT, D), lambda t, f: (t, 0)),
            pl.BlockSpec((TFS, D), lambda t, f: (f, 0)),
            pl.BlockSpec((TFS, D), lambda t, f: (nfs + f, 0)),
            pl.BlockSpec((D, TFS), lambda t, f: (0, f)),
        ],
        out_specs=pl.BlockSpec((TT, D), lambda t, f: (t, 0)),
        out_shape=jax.ShapeDtypeStruct((T, D), jnp.float32),
        compiler_params=pltpu.CompilerParams(
            dimension_semantics=("parallel", "arbitrary")),
    )(x, routed, w13_shared, w13_shared, w2_shared)

    return out
